# baseline (speedup 1.0000x reference)
import functools

import jax
import jax.numpy as jnp
from jax import lax
from jax.experimental import pallas as pl
from jax.experimental.pallas import tpu as pltpu
from jax.experimental.pallas import tpu_sc as plsc

N_NODES = 10000
N_EDGES = 320000
D_FEAT = 128
DW = D_FEAT // 2
NUM_CORES = 2
NUM_SUBCORES = 16
NW = NUM_CORES * NUM_SUBCORES
EPW = N_EDGES // NW
CHUNK = 80
NUM_CHUNKS = EPW // CHUNK
GROUPS = CHUNK // 16


def _dot_scores(hpk, src, dst):
    mesh = plsc.VectorSubcoreMesh(core_axis_name="c", subcore_axis_name="s")

    @functools.partial(
        pl.kernel,
        out_type=jax.ShapeDtypeStruct((N_EDGES,), jnp.float32),
        mesh=mesh,
        compiler_params=pltpu.CompilerParams(
            needs_layout_passes=False, use_tc_tiling_on_sc=False),
        scratch_types=[
            pltpu.VMEM((EPW,), jnp.int32),
            pltpu.VMEM((EPW,), jnp.int32),
            pltpu.VMEM((4, CHUNK, DW), jnp.int32),
            pltpu.VMEM((4, CHUNK, DW), jnp.int32),
            pltpu.VMEM((EPW,), jnp.float32),
            pltpu.SemaphoreType.DMA,
            pltpu.SemaphoreType.DMA,
            pltpu.SemaphoreType.DMA,
            pltpu.SemaphoreType.DMA,
            pltpu.SemaphoreType.DMA,
        ],
    )
    def scores_kernel(h_hbm, src_hbm, dst_hbm, out_hbm,
                      idx_s, idx_d, rows_a, rows_b, scores, sem0, sem1,
                      sem2, sem3, sem_idx):
        wid = lax.axis_index("s") * NUM_CORES + lax.axis_index("c")
        wbase = pl.multiple_of(wid * EPW, 8)
        sems = (sem0, sem1, sem2, sem3)
        lane = lax.iota(jnp.int32, 16)
        rid = [g * 16 + lane for g in range(GROUPS)]
        zero = jnp.zeros((16,), jnp.float32)

        cp_s = pltpu.async_copy(src_hbm.at[pl.ds(wbase, EPW)], idx_s, sem_idx)
        cp_d = pltpu.async_copy(dst_hbm.at[pl.ds(wbase, EPW)], idx_d, sem_idx)
        cp_s.wait()
        cp_d.wait()

        def issue(i, p):
            off = pl.multiple_of(i * CHUNK, 8)
            pltpu.async_copy(
                h_hbm.at[idx_s.at[pl.ds(off, CHUNK)]], rows_a.at[p], sems[p])
            pltpu.async_copy(
                h_hbm.at[idx_d.at[pl.ds(off, CHUNK)]], rows_b.at[p], sems[p])

        def wait(i, p):
            off = pl.multiple_of(i * CHUNK, 8)
            pltpu.make_async_copy(
                h_hbm.at[idx_s.at[pl.ds(off, CHUNK)]], rows_a.at[p],
                sems[p]).wait()
            pltpu.make_async_copy(
                h_hbm.at[idx_d.at[pl.ds(off, CHUNK)]], rows_b.at[p],
                sems[p]).wait()

        def compute(i, p):
            off = i * CHUNK
            ra = rows_a.at[p]
            rb = rows_b.at[p]

            himask = jnp.full((16,), -65536, jnp.int32)

            def unpack2(v_i32):
                lo = plsc.bitcast(v_i32 << 16, jnp.float32)
                hi = plsc.bitcast(v_i32 & himask, jnp.float32)
                return lo, hi

            def d_body(j, accs):
                accs = list(accs)
                for k in range(2):
                    w = j * 2 + k
                    col = (lane + w) & (DW - 1)
                    for g in range(GROUPS):
                        va = plsc.load_gather(ra, [rid[g], col])
                        vb = plsc.load_gather(rb, [rid[g], col])
                        alo, ahi = unpack2(va)
                        blo, bhi = unpack2(vb)
                        accs[g] = accs[g] + alo * blo + ahi * bhi
                return tuple(accs)

            accs = lax.fori_loop(0, DW // 2, d_body, (zero,) * GROUPS)
            for g in range(GROUPS):
                scores[pl.ds(off + g * 16, 16)] = accs[g]

        for t in range(4):
            issue(t, t)

        def body4(j, carry):
            for t in range(4):
                i = 4 * j + t
                wait(i, t)
                compute(i, t)
                if t == 0:
                    issue(i + 4, t)
                else:
                    @pl.when(j < (NUM_CHUNKS - 1) // 4 - 1)
                    def _():
                        issue(i + 4, t)
            return carry

        lax.fori_loop(0, (NUM_CHUNKS - 1) // 4, body4, 0)
        for t in range(NUM_CHUNKS % 4):
            last = NUM_CHUNKS - (NUM_CHUNKS % 4) + t
            wait(last, t)
            compute(last, t)

        pltpu.sync_copy(scores, out_hbm.at[pl.ds(wbase, EPW)])

    return scores_kernel(hpk, src, dst)


def kernel(h, edge_index):
    hb = h.astype(jnp.bfloat16).reshape(N_NODES, DW, 2)
    hpk = jax.lax.bitcast_convert_type(hb, jnp.int32)
    src = edge_index[0].astype(jnp.int32)
    dst = edge_index[1].astype(jnp.int32)
    return _dot_scores(hpk, src, dst)


# speedup vs baseline: 1.0019x; 1.0019x over previous
import functools

import jax
import jax.numpy as jnp
from jax import lax
from jax.experimental import pallas as pl
from jax.experimental.pallas import tpu as pltpu
from jax.experimental.pallas import tpu_sc as plsc

N_NODES = 10000
N_EDGES = 320000
D_FEAT = 128
DW = D_FEAT // 2
NUM_CORES = 2
NUM_SUBCORES = 16
NW = NUM_CORES * NUM_SUBCORES
EPW = N_EDGES // NW
CHUNK = 80
NUM_CHUNKS = EPW // CHUNK
GROUPS = CHUNK // 16


def _dot_scores(hpk, src, dst):
    mesh = plsc.VectorSubcoreMesh(core_axis_name="c", subcore_axis_name="s")

    @functools.partial(
        pl.kernel,
        out_type=jax.ShapeDtypeStruct((N_EDGES,), jnp.float32),
        mesh=mesh,
        compiler_params=pltpu.CompilerParams(
            needs_layout_passes=False, use_tc_tiling_on_sc=False),
        scratch_types=[
            pltpu.VMEM((EPW,), jnp.int32),
            pltpu.VMEM((EPW,), jnp.int32),
            pltpu.VMEM((4, CHUNK, DW), jnp.int32),
            pltpu.VMEM((4, CHUNK, DW), jnp.int32),
            pltpu.VMEM((EPW,), jnp.float32),
            pltpu.SemaphoreType.DMA,
            pltpu.SemaphoreType.DMA,
            pltpu.SemaphoreType.DMA,
            pltpu.SemaphoreType.DMA,
            pltpu.SemaphoreType.DMA,
        ],
    )
    def scores_kernel(h_hbm, src_hbm, dst_hbm, out_hbm,
                      idx_s, idx_d, rows_a, rows_b, scores, sem0, sem1,
                      sem2, sem3, sem_idx):
        wid = lax.axis_index("s") * NUM_CORES + lax.axis_index("c")
        wbase = pl.multiple_of(wid * EPW, 8)
        sems = (sem0, sem1, sem2, sem3)
        lane = lax.iota(jnp.int32, 16)
        rid = [g * 16 + lane for g in range(GROUPS)]
        zero = jnp.zeros((16,), jnp.float32)

        head = 4 * CHUNK
        cp_s = pltpu.async_copy(
            src_hbm.at[pl.ds(wbase, head)], idx_s.at[pl.ds(0, head)], sem_idx)
        cp_d = pltpu.async_copy(
            dst_hbm.at[pl.ds(wbase, head)], idx_d.at[pl.ds(0, head)], sem_idx)
        cp_s.wait()
        cp_d.wait()

        def issue(i, p):
            off = pl.multiple_of(i * CHUNK, 8)
            pltpu.async_copy(
                h_hbm.at[idx_s.at[pl.ds(off, CHUNK)]], rows_a.at[p], sems[p])
            pltpu.async_copy(
                h_hbm.at[idx_d.at[pl.ds(off, CHUNK)]], rows_b.at[p], sems[p])

        def wait(i, p):
            off = pl.multiple_of(i * CHUNK, 8)
            pltpu.make_async_copy(
                h_hbm.at[idx_s.at[pl.ds(off, CHUNK)]], rows_a.at[p],
                sems[p]).wait()
            pltpu.make_async_copy(
                h_hbm.at[idx_d.at[pl.ds(off, CHUNK)]], rows_b.at[p],
                sems[p]).wait()

        def compute(i, p):
            off = i * CHUNK
            ra = rows_a.at[p]
            rb = rows_b.at[p]

            himask = jnp.full((16,), -65536, jnp.int32)

            def unpack2(v_i32):
                lo = plsc.bitcast(v_i32 << 16, jnp.float32)
                hi = plsc.bitcast(v_i32 & himask, jnp.float32)
                return lo, hi

            def d_body(j, accs):
                accs = list(accs)
                for k in range(2):
                    w = j * 2 + k
                    col = (lane + w) & (DW - 1)
                    for g in range(GROUPS):
                        va = plsc.load_gather(ra, [rid[g], col])
                        vb = plsc.load_gather(rb, [rid[g], col])
                        alo, ahi = unpack2(va)
                        blo, bhi = unpack2(vb)
                        accs[g] = accs[g] + alo * blo + ahi * bhi
                return tuple(accs)

            accs = lax.fori_loop(0, DW // 2, d_body, (zero,) * GROUPS)
            for g in range(GROUPS):
                scores[pl.ds(off + g * 16, 16)] = accs[g]

        for t in range(4):
            issue(t, t)

        cp_s2 = pltpu.async_copy(
            src_hbm.at[pl.ds(wbase + head, EPW - head)],
            idx_s.at[pl.ds(head, EPW - head)], sem_idx)
        cp_d2 = pltpu.async_copy(
            dst_hbm.at[pl.ds(wbase + head, EPW - head)],
            idx_d.at[pl.ds(head, EPW - head)], sem_idx)
        cp_s2.wait()
        cp_d2.wait()

        def body4(j, carry):
            for t in range(4):
                i = 4 * j + t
                wait(i, t)
                compute(i, t)
                if t == 0:
                    issue(i + 4, t)
                else:
                    @pl.when(j < (NUM_CHUNKS - 1) // 4 - 1)
                    def _():
                        issue(i + 4, t)
            return carry

        lax.fori_loop(0, (NUM_CHUNKS - 1) // 4, body4, 0)
        for t in range(NUM_CHUNKS % 4):
            last = NUM_CHUNKS - (NUM_CHUNKS % 4) + t
            wait(last, t)
            compute(last, t)

        pltpu.sync_copy(scores, out_hbm.at[pl.ds(wbase, EPW)])

    return scores_kernel(hpk, src, dst)


def kernel(h, edge_index):
    hb = h.astype(jnp.bfloat16).reshape(N_NODES, DW, 2)
    hpk = jax.lax.bitcast_convert_type(hb, jnp.int32)
    src = edge_index[0].astype(jnp.int32)
    dst = edge_index[1].astype(jnp.int32)
    return _dot_scores(hpk, src, dst)
